# baseline (speedup 1.0000x reference)
import functools

import jax
import jax.numpy as jnp
from jax import lax
from jax.experimental import pallas as pl
from jax.experimental.pallas import tpu as pltpu
from jax.experimental.pallas import tpu_sc as plsc

VOCAB = 100000
DIMS = 64
BATCH = 1024
CTX = 20

NC = 2
NS = 16
LANES = 16
NW = NC * NS
B_PER_W = BATCH // NW
IDX_PER_W = B_PER_W * CTX
IDX_MINOR = 128
KROWS = IDX_PER_W // IDX_MINOR

VBLK = 512
NVBLK = (VOCAB + VBLK - 1) // VBLK


def _sc_embed_body(idx_hbm, table_hbm, out_hbm, idx_v, rows_v, out_v, sem):
    c = lax.axis_index("c")
    s = lax.axis_index("s")
    wid = s * NC + c

    pltpu.sync_copy(idx_hbm.at[wid], idx_v)

    copies = []
    for j in range(KROWS):
        copies.append(
            pltpu.async_copy(
                table_hbm.at[idx_v.at[j]],
                rows_v.at[pl.ds(j * IDX_MINOR, IDX_MINOR)],
                sem,
            )
        )
    for cp in copies:
        cp.wait()

    def body(e, carry):
        base = e * CTX
        for v in range(DIMS // LANES):
            acc = rows_v[base, pl.ds(v * LANES, LANES)]
            for k in range(1, CTX):
                acc = acc + rows_v[base + k, pl.ds(v * LANES, LANES)]
            out_v[e, pl.ds(v * LANES, LANES)] = acc
        return carry

    lax.fori_loop(0, B_PER_W, body, 0)

    pltpu.sync_copy(out_v, out_hbm.at[pl.ds(wid * B_PER_W, B_PER_W)])


_sc_embed = functools.partial(
    pl.kernel,
    mesh=plsc.VectorSubcoreMesh(core_axis_name="c", subcore_axis_name="s"),
    out_type=jax.ShapeDtypeStruct((BATCH, DIMS), jnp.float32),
    scratch_types=[
        pltpu.VMEM((KROWS, IDX_MINOR), jnp.int32),
        pltpu.VMEM((IDX_PER_W, DIMS), jnp.float32),
        pltpu.VMEM((B_PER_W, DIMS), jnp.float32),
        pltpu.SemaphoreType.DMA,
    ],
    compiler_params=pltpu.CompilerParams(use_tc_tiling_on_sc=False),
)(_sc_embed_body)


def _tc_matmul_body(emb_ref, w_ref, b_ref, out_ref):
    out_ref[...] = (
        lax.dot_general(
            emb_ref[...],
            w_ref[...],
            dimension_numbers=(((1,), (1,)), ((), ())),
            preferred_element_type=jnp.float32,
        )
        + b_ref[...]
    )


def _tc_matmul(embeds, W, b2d):
    return pl.pallas_call(
        _tc_matmul_body,
        grid=(NVBLK,),
        in_specs=[
            pl.BlockSpec((BATCH, DIMS), lambda i: (0, 0)),
            pl.BlockSpec((VBLK, DIMS), lambda i: (i, 0)),
            pl.BlockSpec((1, VBLK), lambda i: (0, i)),
        ],
        out_specs=pl.BlockSpec((BATCH, VBLK), lambda i: (0, i)),
        out_shape=jax.ShapeDtypeStruct((BATCH, VOCAB), jnp.float32),
        compiler_params=pltpu.CompilerParams(
            dimension_semantics=("arbitrary",),
        ),
    )(embeds, W, b2d)


def kernel(inputs, emb_table, W, b):
    idx = inputs.astype(jnp.int32).reshape(NW, KROWS, IDX_MINOR)
    embeds = _sc_embed(idx, emb_table)
    return _tc_matmul(embeds, W, b.reshape(1, VOCAB))


# speedup vs baseline: 1.1321x; 1.1321x over previous
import functools

import jax
import jax.numpy as jnp
from jax import lax
from jax.experimental import pallas as pl
from jax.experimental.pallas import tpu as pltpu
from jax.experimental.pallas import tpu_sc as plsc

VOCAB = 100000
DIMS = 64
BATCH = 1024
CTX = 20

NC = 2
NS = 16
LANES = 16
NW = NC * NS
B_PER_W = BATCH // NW
IDX_PER_W = B_PER_W * CTX
IDX_MINOR = 128
KROWS = IDX_PER_W // IDX_MINOR

VBLK = 2048
NVBLK = (VOCAB + VBLK - 1) // VBLK


def _sc_embed_body(idx_hbm, table_hbm, out_hbm, idx_v, rows_v, out_v, sem):
    c = lax.axis_index("c")
    s = lax.axis_index("s")
    wid = s * NC + c

    pltpu.sync_copy(idx_hbm.at[wid], idx_v)

    copies = []
    for j in range(KROWS):
        copies.append(
            pltpu.async_copy(
                table_hbm.at[idx_v.at[j]],
                rows_v.at[pl.ds(j * IDX_MINOR, IDX_MINOR)],
                sem,
            )
        )
    for cp in copies:
        cp.wait()

    def body(e, carry):
        base = e * CTX
        for v in range(DIMS // LANES):
            acc = rows_v[base, pl.ds(v * LANES, LANES)]
            for k in range(1, CTX):
                acc = acc + rows_v[base + k, pl.ds(v * LANES, LANES)]
            out_v[e, pl.ds(v * LANES, LANES)] = acc
        return carry

    lax.fori_loop(0, B_PER_W, body, 0)

    pltpu.sync_copy(out_v, out_hbm.at[pl.ds(wid * B_PER_W, B_PER_W)])


_sc_embed = functools.partial(
    pl.kernel,
    mesh=plsc.VectorSubcoreMesh(core_axis_name="c", subcore_axis_name="s"),
    out_type=jax.ShapeDtypeStruct((BATCH, DIMS), jnp.float32),
    scratch_types=[
        pltpu.VMEM((KROWS, IDX_MINOR), jnp.int32),
        pltpu.VMEM((IDX_PER_W, DIMS), jnp.float32),
        pltpu.VMEM((B_PER_W, DIMS), jnp.float32),
        pltpu.SemaphoreType.DMA,
    ],
    compiler_params=pltpu.CompilerParams(use_tc_tiling_on_sc=False),
)(_sc_embed_body)


def _tc_matmul_body(emb_ref, w_ref, b_ref, out_ref):
    out_ref[...] = (
        lax.dot_general(
            emb_ref[...],
            w_ref[...],
            dimension_numbers=(((1,), (1,)), ((), ())),
            preferred_element_type=jnp.float32,
        )
        + b_ref[...]
    )


def _tc_matmul(embeds, W, b2d):
    return pl.pallas_call(
        _tc_matmul_body,
        grid=(NVBLK,),
        in_specs=[
            pl.BlockSpec((BATCH, DIMS), lambda i: (0, 0)),
            pl.BlockSpec((VBLK, DIMS), lambda i: (i, 0)),
            pl.BlockSpec((1, VBLK), lambda i: (0, i)),
        ],
        out_specs=pl.BlockSpec((BATCH, VBLK), lambda i: (0, i)),
        out_shape=jax.ShapeDtypeStruct((BATCH, VOCAB), jnp.float32),
        compiler_params=pltpu.CompilerParams(
            dimension_semantics=("parallel",),
        ),
    )(embeds, W, b2d)


def kernel(inputs, emb_table, W, b):
    idx = inputs.astype(jnp.int32).reshape(NW, KROWS, IDX_MINOR)
    embeds = _sc_embed(idx, emb_table)
    return _tc_matmul(embeds, W, b.reshape(1, VOCAB))


# speedup vs baseline: 1.2652x; 1.1176x over previous
import functools

import jax
import jax.numpy as jnp
from jax import lax
from jax.experimental import pallas as pl
from jax.experimental.pallas import tpu as pltpu
from jax.experimental.pallas import tpu_sc as plsc

VOCAB = 100000
DIMS = 64
BATCH = 1024
CTX = 20

NC = 2
NS = 16
LANES = 16
NW = NC * NS
B_PER_W = BATCH // NW
IDX_PER_W = B_PER_W * CTX
IDX_MINOR = 128
KROWS = IDX_PER_W // IDX_MINOR

VBLK = 1408
NVBLK = 71
VMAIN = NVBLK * VBLK
VTAIL = VOCAB - VMAIN
NBUF = 3


def _sc_embed_body(idx_hbm, table_hbm, out_hbm, idx_v, rows_v, out_v, sem):
    c = lax.axis_index("c")
    s = lax.axis_index("s")
    wid = s * NC + c

    pltpu.sync_copy(idx_hbm.at[wid], idx_v)

    copies = []
    for j in range(KROWS):
        copies.append(
            pltpu.async_copy(
                table_hbm.at[idx_v.at[j]],
                rows_v.at[pl.ds(j * IDX_MINOR, IDX_MINOR)],
                sem,
            )
        )
    for cp in copies:
        cp.wait()

    def body(e, carry):
        base = e * CTX
        for v in range(DIMS // LANES):
            acc = rows_v[base, pl.ds(v * LANES, LANES)]
            for k in range(1, CTX):
                acc = acc + rows_v[base + k, pl.ds(v * LANES, LANES)]
            out_v[e, pl.ds(v * LANES, LANES)] = acc
        return carry

    lax.fori_loop(0, B_PER_W, body, 0)

    pltpu.sync_copy(out_v, out_hbm.at[pl.ds(wid * B_PER_W, B_PER_W)])


_sc_embed = functools.partial(
    pl.kernel,
    mesh=plsc.VectorSubcoreMesh(core_axis_name="c", subcore_axis_name="s"),
    out_type=jax.ShapeDtypeStruct((BATCH, DIMS), jnp.float32),
    scratch_types=[
        pltpu.VMEM((KROWS, IDX_MINOR), jnp.int32),
        pltpu.VMEM((IDX_PER_W, DIMS), jnp.float32),
        pltpu.VMEM((B_PER_W, DIMS), jnp.float32),
        pltpu.SemaphoreType.DMA,
    ],
    compiler_params=pltpu.CompilerParams(use_tc_tiling_on_sc=False),
)(_sc_embed_body)


def _dot_nt(emb, w, bias):
    return (
        lax.dot_general(
            emb,
            w,
            dimension_numbers=(((1,), (1,)), ((), ())),
            preferred_element_type=jnp.float32,
        )
        + bias
    )


def _tc_matmul_body(emb_ref, w_ref, b_ref, wt_ref, bt_ref, out_hbm,
                    tail_ref, acc_ref, sems):
    i = pl.program_id(0)
    slot = lax.rem(i, NBUF)

    @pl.when(i >= NBUF)
    def _():
        pltpu.make_async_copy(
            acc_ref.at[slot],
            out_hbm.at[:, pl.ds(0, VBLK)],
            sems.at[slot],
        ).wait()

    acc_ref[slot] = _dot_nt(emb_ref[...], w_ref[...], b_ref[...])

    pltpu.make_async_copy(
        acc_ref.at[slot],
        out_hbm.at[:, pl.ds(i * VBLK, VBLK)],
        sems.at[slot],
    ).start()

    @pl.when(i == NVBLK - 1)
    def _():
        tail_ref[...] = _dot_nt(emb_ref[...], wt_ref[...], bt_ref[...])
        for d in range(NBUF):
            pltpu.make_async_copy(
                acc_ref.at[lax.rem(i - d + NBUF, NBUF)],
                out_hbm.at[:, pl.ds(0, VBLK)],
                sems.at[lax.rem(i - d + NBUF, NBUF)],
            ).wait()


def _tc_matmul(embeds, W, b2d, W_tail, b_tail):
    return pl.pallas_call(
        _tc_matmul_body,
        grid=(NVBLK,),
        in_specs=[
            pl.BlockSpec((BATCH, DIMS), lambda i: (0, 0)),
            pl.BlockSpec((VBLK, DIMS), lambda i: (i, 0)),
            pl.BlockSpec((1, VBLK), lambda i: (0, i)),
            pl.BlockSpec((VTAIL, DIMS), lambda i: (0, 0)),
            pl.BlockSpec((1, VTAIL), lambda i: (0, 0)),
        ],
        out_specs=[
            pl.BlockSpec(memory_space=pl.ANY),
            pl.BlockSpec((BATCH, VTAIL), lambda i: (0, 0)),
        ],
        out_shape=[
            jax.ShapeDtypeStruct((BATCH, VOCAB), jnp.float32),
            jax.ShapeDtypeStruct((BATCH, VTAIL), jnp.float32),
        ],
        scratch_shapes=[
            pltpu.VMEM((NBUF, BATCH, VBLK), jnp.float32),
            pltpu.SemaphoreType.DMA((NBUF,)),
        ],
        compiler_params=pltpu.CompilerParams(
            dimension_semantics=("arbitrary",),
            vmem_limit_bytes=100 * 1024 * 1024,
        ),
    )(embeds, W, b2d, W_tail, b_tail)


def kernel(inputs, emb_table, W, b):
    idx = inputs.astype(jnp.int32).reshape(NW, KROWS, IDX_MINOR)
    embeds = _sc_embed(idx, emb_table)
    W_tail = lax.slice(W, (VMAIN, 0), (VOCAB, DIMS))
    b_tail = lax.slice(b, (VMAIN,), (VOCAB,)).reshape(1, VTAIL)
    out_main, out_tail = _tc_matmul(embeds, W, b.reshape(1, VOCAB),
                                    W_tail, b_tail)
    return lax.dynamic_update_slice(out_main, out_tail, (0, VMAIN))
